# baseline (speedup 1.0000x reference)
import functools

import jax
import jax.numpy as jnp
from jax import lax
from jax.experimental import pallas as pl
from jax.experimental.pallas import tpu as pltpu
from jax.experimental.pallas import tpu_sc as plsc

_BATCH = 4096
_SEQ = 200
_EMBED = 64
_N = _BATCH * _SEQ
_NC = 2
_NS = 16
_NW = _NC * _NS
_N_W = _N // _NW
_CHUNK = _SEQ
_NCHUNK = _N_W // _CHUNK
_LANES = 16


def _make_kernel():
  mesh = plsc.VectorSubcoreMesh(core_axis_name="c", subcore_axis_name="s")

  @functools.partial(
      pl.kernel,
      mesh=mesh,
      out_type=jax.ShapeDtypeStruct((_N, _EMBED), jnp.float32),
      compiler_params=pltpu.CompilerParams(use_tc_tiling_on_sc=False),
      scratch_types=[
          pltpu.VMEM((_CHUNK,), jnp.int32),
          pltpu.VMEM((_CHUNK, _EMBED), jnp.float32),
          pltpu.VMEM((_SEQ, _EMBED), jnp.float32),
          pltpu.SemaphoreType.DMA,
      ],
  )
  def _embed(tokens_hbm, table_hbm, pos_hbm, out_hbm, idx_v, rows_v, pos_v,
             sem):
    wid = lax.axis_index("s") * _NC + lax.axis_index("c")
    base = wid * _N_W

    pltpu.sync_copy(pos_hbm, pos_v)

    def add_pos_row(j, carry):
      for q in range(_EMBED // _LANES):
        sl = pl.ds(q * _LANES, _LANES)
        rows_v[j, sl] = rows_v[j, sl] + pos_v[j, sl]
      return carry

    def chunk_body(c, carry):
      rbase = base + c * _CHUNK
      pltpu.sync_copy(tokens_hbm.at[pl.ds(rbase, _CHUNK)], idx_v)
      pltpu.async_copy(table_hbm.at[idx_v], rows_v, sem).wait()
      lax.fori_loop(0, _CHUNK, add_pos_row, 0, unroll=False)
      pltpu.sync_copy(rows_v, out_hbm.at[pl.ds(rbase, _CHUNK)])
      return carry

    lax.fori_loop(0, _NCHUNK, chunk_body, 0, unroll=False)

  return _embed


_embed_kernel = _make_kernel()


def kernel(tokens, token_table, pos_table):
  flat_tokens = tokens.reshape(_N).astype(jnp.int32)
  out = _embed_kernel(flat_tokens, token_table, pos_table)
  return out.reshape(_BATCH, _SEQ, _EMBED)
